# baseline (speedup 1.0000x reference)
import functools

import jax
import jax.numpy as jnp
from jax import lax
from jax.experimental import pallas as pl
from jax.experimental.pallas import tpu as pltpu
from jax.experimental.pallas import tpu_sc as plsc

B, S, F = 1024, 200, 32
H = 256
NUM_COLS = 8
CAT_COLS = F - NUM_COLS
P = H // F
DICT = 65536
N_IDX = B * S * CAT_COLS

NC, NS = 2, 16
NW = NC * NS
PER_W = N_IDX // NW
CHUNK = 128
K = 16
GROUPS = PER_W // (CHUNK * K)

_mesh = plsc.VectorSubcoreMesh(core_axis_name="c", subcore_axis_name="s")

VTILES = DICT // 128
TILES_TOTAL = CAT_COLS * VTILES
TPW = TILES_TOTAL // NW


@functools.partial(
    pl.kernel,
    mesh=_mesh,
    compiler_params=pltpu.CompilerParams(
        use_tc_tiling_on_sc=False, needs_layout_passes=False
    ),
    out_type=jax.ShapeDtypeStruct((TILES_TOTAL, 1024), jnp.float32),
    scratch_types=[
        pltpu.VMEM((256, 128), jnp.float32),
        pltpu.VMEM((256, 128), jnp.float32),
        pltpu.VMEM((32, 1024), jnp.float32),
        pltpu.SemaphoreType.DMA,
        pltpu.SemaphoreType.DMA,
    ],
)
def _sc_tablefmt(tbl_hbm, out_hbm, big0, big1, out_v, sem0, sem1):
    wid = lax.axis_index("s") * NC + lax.axis_index("c")
    iota = lax.iota(jnp.int32, 16)
    p_vec = jnp.bitwise_and(iota, P - 1)
    l_base = jnp.right_shift(iota, 3)
    row0 = wid * TPW * P
    NCH = TPW // 32

    def chunk_in(g, big, sem):
        g = jnp.minimum(g, NCH - 1)
        return pltpu.async_copy(tbl_hbm.at[pl.ds(row0 + g * 256, 256)], big, sem)

    def compute(g, big):
        def tile_body(k, carry):
            base = k * P + p_vec
            for v0 in range(0, 64, 4):
                xs = [plsc.load_gather(big, [base, l_base + 2 * (v0 + u)])
                      for u in range(4)]
                for u in range(4):
                    out_v[k, pl.ds(16 * (v0 + u), 16)] = xs[u]
            return carry
        lax.fori_loop(0, 32, tile_body, 0)
        pltpu.sync_copy(out_v, out_hbm.at[pl.ds(wid * TPW + g * 32, 32)])

    chunk_in(0, big0, sem0)

    def body(i, carry):
        pltpu.make_async_copy(tbl_hbm.at[pl.ds(row0, 256)], big0, sem0).wait()
        chunk_in(2 * i + 1, big1, sem1)
        compute(2 * i, big0)
        chunk_in(2 * i + 2, big0, sem0)
        pltpu.make_async_copy(tbl_hbm.at[pl.ds(row0, 256)], big1, sem1).wait()
        compute(2 * i + 1, big1)
        return carry

    lax.fori_loop(0, NCH // 2, body, 0)
    pltpu.make_async_copy(tbl_hbm.at[pl.ds(row0, 256)], big0, sem0).wait()


SS = 4
SPAD = ((S + 1 + SS - 1) // SS) * SS
ROWS_PER_S = B * CAT_COLS

SEG_TOK = 104
SEG_POS = SEG_TOK + SS
SEG_ROWS = SEG_POS * ROWS_PER_S
TOK0_B = S - SEG_TOK
NBLK_A = SEG_TOK // SS


def _make_gather(tok0):
    n_seg = SEG_TOK * ROWS_PER_S
    per_w = n_seg // NW
    groups = per_w // (K * CHUNK)

    @functools.partial(
        pl.kernel,
        mesh=_mesh,
        compiler_params=pltpu.CompilerParams(use_tc_tiling_on_sc=False),
        out_type=jax.ShapeDtypeStruct((SEG_ROWS, P), jnp.float32),
        scratch_types=[
            pltpu.VMEM((K, CHUNK), jnp.int32),
            pltpu.VMEM((K, CHUNK), jnp.int32),
            pltpu.VMEM((K * CHUNK, P), jnp.float32),
            pltpu.VMEM((K * CHUNK, P), jnp.float32),
            pltpu.SemaphoreType.DMA,
            pltpu.SemaphoreType.DMA,
            pltpu.SemaphoreType.DMA,
            pltpu.SemaphoreType.DMA,
            pltpu.SemaphoreType.DMA,
            pltpu.SemaphoreType.DMA,
        ],
    )
    def _gather(idx_hbm, table_hbm, out_hbm, idx0, idx1, rows0, rows1,
                semi0, semi1, semg0, semg1, semo0, semo1):
        wid = lax.axis_index("s") * NC + lax.axis_index("c")
        ibase = tok0 * (ROWS_PER_S // CHUNK) + wid * (per_w // CHUNK)
        obase = ROWS_PER_S + wid * per_w

        def idx_in(g, idx_v, sem):
            g = jnp.minimum(g, groups - 1)
            return pltpu.async_copy(
                idx_hbm.at[pl.ds(ibase + g * K, K)], idx_v, sem
            )

        def idx_wait(idx_v, sem):
            pltpu.make_async_copy(
                idx_hbm.at[pl.ds(ibase, K)], idx_v, sem
            ).wait()

        def fire(idx_v, rows_v, sem):
            for j in range(K):
                pltpu.async_copy(
                    table_hbm.at[idx_v.at[j]],
                    rows_v.at[pl.ds(j * CHUNK, CHUNK)],
                    sem,
                )

        def drain(idx_v, rows_v, sem):
            for j in range(K):
                pltpu.make_async_copy(
                    table_hbm.at[idx_v.at[j]],
                    rows_v.at[pl.ds(j * CHUNK, CHUNK)],
                    sem,
                ).wait()

        def out_start(g, rows_v, sem):
            pltpu.async_copy(
                rows_v,
                out_hbm.at[pl.ds(obase + g * K * CHUNK, K * CHUNK)],
                sem,
            )

        def out_wait(rows_v, sem):
            pltpu.make_async_copy(
                rows_v, out_hbm.at[pl.ds(obase, K * CHUNK)], sem
            ).wait()

        pltpu.sync_copy(idx_hbm.at[pl.ds(ibase, K)], idx0)
        fire(idx0, rows0, semg0)
        idx_in(1, idx1, semi1)
        idx_wait(idx1, semi1)
        fire(idx1, rows1, semg1)
        drain(idx0, rows0, semg0)
        out_start(0, rows0, semo0)
        idx_in(2, idx0, semi0)

        def body(i, carry):
            idx_wait(idx0, semi0)
            out_wait(rows0, semo0)
            fire(idx0, rows0, semg0)
            drain(idx1, rows1, semg1)
            out_start(2 * i - 1, rows1, semo1)
            idx_in(2 * i + 1, idx1, semi1)
            idx_wait(idx1, semi1)
            out_wait(rows1, semo1)
            fire(idx1, rows1, semg1)
            drain(idx0, rows0, semg0)
            out_start(2 * i, rows0, semo0)
            idx_in(2 * i + 2, idx0, semi0)
            return carry

        lax.fori_loop(1, (groups - 1) // 2, body, 0)
        idx_wait(idx0, semi0)
        out_wait(rows0, semo0)
        fire(idx0, rows0, semg0)
        drain(idx1, rows1, semg1)
        out_start(groups - 2, rows1, semo1)
        drain(idx0, rows0, semg0)
        out_start(groups - 1, rows0, semo0)
        out_wait(rows1, semo1)
        out_wait(rows0, semo0)

    return _gather


_sc_gather_a = _make_gather(0)
_sc_gather_b = _make_gather(TOK0_B)


def _tc_body(inp_ref, feata_ref, featb_ref, nw_ref, wnr_ref, wc_ref, pos_ref,
             cls_ref, gam_ref, bet_ref, out_ref):
    m = lax.dot_general(
        nw_ref[...], wnr_ref[...],
        dimension_numbers=(((1,), (1,)), ((0,), (0,))),
        preferred_element_type=jnp.float32,
    )
    feat = lax.cond(
        pl.program_id(0) < NBLK_A,
        lambda: feata_ref[...],
        lambda: featb_ref[...],
    )
    emb = lax.dot_general(
        feat, wc_ref[...],
        dimension_numbers=(((2,), (0,)), ((), ())),
        preferred_element_type=jnp.float32,
    )
    emb = emb + lax.dot_general(
        inp_ref[...], m,
        dimension_numbers=(((1,), (0,)), ((), ())),
        preferred_element_type=jnp.float32,
    )
    hid = emb + pos_ref[0][:, None, :]
    mean = jnp.mean(hid, axis=-1, keepdims=True)
    var = jnp.mean((hid - mean) ** 2, axis=-1, keepdims=True)
    body = (hid - mean) * lax.rsqrt(var + 1e-5) * gam_ref[...] + bet_ref[...]
    out_ref[...] = body

    @pl.when(pl.program_id(0) == 0)
    def _cls_row():
        row0 = cls_ref[...] + pos_ref[0, 0:1, :]
        m0 = jnp.mean(row0, axis=-1, keepdims=True)
        v0 = jnp.mean((row0 - m0) ** 2, axis=-1, keepdims=True)
        r0 = (row0 - m0) * lax.rsqrt(v0 + 1e-5) * gam_ref[...] + bet_ref[...]
        out_ref[0:1, :, :] = jnp.broadcast_to(r0[:, None, :], (1, B, H))


def _tc_dense(inp, feat_a, feat_b, num_w, wn_r, wc, pos, cls2d, gam, bet):
    blkb0 = TOK0_B // SS
    return pl.pallas_call(
        _tc_body,
        grid=(SPAD // SS,),
        in_specs=[
            pl.BlockSpec((SS, NUM_COLS, B), lambda i: (i, 0, 0)),
            pl.BlockSpec((SS, B, CAT_COLS * P),
                         lambda i: (jnp.minimum(i, NBLK_A), 0, 0)),
            pl.BlockSpec((SS, B, CAT_COLS * P),
                         lambda i: (jnp.maximum(i, blkb0) - blkb0, 0, 0)),
            pl.BlockSpec((NUM_COLS, P), lambda i: (0, 0)),
            pl.BlockSpec((NUM_COLS, P, H), lambda i: (0, 0, 0)),
            pl.BlockSpec((CAT_COLS * P, H), lambda i: (0, 0)),
            pl.BlockSpec((1, SS, H), lambda i: (i, 0, 0)),
            pl.BlockSpec((1, H), lambda i: (0, 0)),
            pl.BlockSpec((1, H), lambda i: (0, 0)),
            pl.BlockSpec((1, H), lambda i: (0, 0)),
        ],
        out_specs=pl.BlockSpec((SS, B, H), lambda i: (i, 0, 0)),
        out_shape=jax.ShapeDtypeStruct((S + 1, B, H), jnp.float32),
    )(inp, feat_a, feat_b, num_w, wn_r, wc, pos, cls2d, gam, bet)


def kernel(input, cls_token, pos_table, cat_tables, num_weights, proj_w,
           gamma, beta):
    cat_idx = input[:, :, NUM_COLS:].astype(jnp.int32)
    offs = jnp.arange(CAT_COLS, dtype=jnp.int32) * DICT
    idx = (cat_idx + offs).transpose(1, 0, 2).reshape(N_IDX // CHUNK, CHUNK)
    tbl_phys = cat_tables.reshape(CAT_COLS, VTILES, 128, P)
    tbl_phys = tbl_phys.transpose(0, 1, 3, 2).reshape(TILES_TOTAL * P, 128)
    table_fmt = _sc_tablefmt(tbl_phys)
    table = table_fmt.reshape(CAT_COLS * DICT, P)
    feat_a = _sc_gather_a(idx, table).reshape(SEG_POS, B, CAT_COLS * P)
    feat_b = _sc_gather_b(idx, table).reshape(SEG_POS, B, CAT_COLS * P)
    inp_t = input[:, :, :NUM_COLS].transpose(1, 2, 0)
    inp_pad = jnp.pad(inp_t, ((1, SPAD - S - 1), (0, 0), (0, 0)))
    pos_pad = jnp.pad(pos_table, ((0, SPAD - S - 1), (0, 0)))
    pos_pad = pos_pad.reshape(SPAD // SS, SS, H)
    wn_r = proj_w[:, : NUM_COLS * P].reshape(H, NUM_COLS, P).transpose(1, 2, 0)
    wc = proj_w[:, NUM_COLS * P:].T
    out_sm = _tc_dense(inp_pad, feat_a, feat_b, num_weights, wn_r, wc,
                       pos_pad, cls_token.reshape(1, H), gamma.reshape(1, H),
                       beta.reshape(1, H))
    return out_sm.transpose(1, 0, 2)


# speedup vs baseline: 1.0615x; 1.0615x over previous
import functools

import jax
import jax.numpy as jnp
from jax import lax
from jax.experimental import pallas as pl
from jax.experimental.pallas import tpu as pltpu
from jax.experimental.pallas import tpu_sc as plsc

B, S, F = 1024, 200, 32
H = 256
NUM_COLS = 8
CAT_COLS = F - NUM_COLS
P = H // F
DICT = 65536
N_IDX = B * S * CAT_COLS

NC, NS = 2, 16
NW = NC * NS
PER_W = N_IDX // NW
CHUNK = 128
K = 16
GROUPS = PER_W // (CHUNK * K)

_mesh = plsc.VectorSubcoreMesh(core_axis_name="c", subcore_axis_name="s")

VTILES = DICT // 128
TILES_TOTAL = CAT_COLS * VTILES
TPW = TILES_TOTAL // NW


@functools.partial(
    pl.kernel,
    mesh=_mesh,
    compiler_params=pltpu.CompilerParams(
        use_tc_tiling_on_sc=False, needs_layout_passes=False
    ),
    out_type=jax.ShapeDtypeStruct((TILES_TOTAL, 1024), jnp.float32),
    scratch_types=[
        pltpu.VMEM((256, 128), jnp.float32),
        pltpu.VMEM((256, 128), jnp.float32),
        pltpu.VMEM((32, 1024), jnp.float32),
        pltpu.SemaphoreType.DMA,
        pltpu.SemaphoreType.DMA,
    ],
)
def _sc_tablefmt(tbl_hbm, out_hbm, big0, big1, out_v, sem0, sem1):
    wid = lax.axis_index("s") * NC + lax.axis_index("c")
    iota = lax.iota(jnp.int32, 16)
    p_vec = jnp.bitwise_and(iota, P - 1)
    l_base = jnp.right_shift(iota, 3)
    row0 = wid * TPW * P
    NCH = TPW // 32

    def chunk_in(g, big, sem):
        g = jnp.minimum(g, NCH - 1)
        return pltpu.async_copy(tbl_hbm.at[pl.ds(row0 + g * 256, 256)], big, sem)

    def compute(g, big):
        def tile_body(k, carry):
            base = k * P + p_vec
            for v0 in range(0, 64, 4):
                xs = [plsc.load_gather(big, [base, l_base + 2 * (v0 + u)])
                      for u in range(4)]
                for u in range(4):
                    out_v[k, pl.ds(16 * (v0 + u), 16)] = xs[u]
            return carry
        lax.fori_loop(0, 32, tile_body, 0)
        pltpu.sync_copy(out_v, out_hbm.at[pl.ds(wid * TPW + g * 32, 32)])

    chunk_in(0, big0, sem0)

    def body(i, carry):
        pltpu.make_async_copy(tbl_hbm.at[pl.ds(row0, 256)], big0, sem0).wait()
        chunk_in(2 * i + 1, big1, sem1)
        compute(2 * i, big0)
        chunk_in(2 * i + 2, big0, sem0)
        pltpu.make_async_copy(tbl_hbm.at[pl.ds(row0, 256)], big1, sem1).wait()
        compute(2 * i + 1, big1)
        return carry

    lax.fori_loop(0, NCH // 2, body, 0)
    pltpu.make_async_copy(tbl_hbm.at[pl.ds(row0, 256)], big0, sem0).wait()


SS = 8
SPAD = ((S + 1 + SS - 1) // SS) * SS
ROWS_PER_S = B * CAT_COLS

SEG_POS = 104
SEG_ROWS = SEG_POS * ROWS_PER_S
NBLK_A = SEG_POS // SS
KSEG = 6


def _make_gather(tok0, ntok, out_shift):
    per_w = ntok * (ROWS_PER_S // NW)
    groups = per_w // (KSEG * CHUNK)
    K = KSEG

    @functools.partial(
        pl.kernel,
        mesh=_mesh,
        compiler_params=pltpu.CompilerParams(use_tc_tiling_on_sc=False),
        out_type=jax.ShapeDtypeStruct((SEG_ROWS, P), jnp.float32),
        scratch_types=[
            pltpu.VMEM((K, CHUNK), jnp.int32),
            pltpu.VMEM((K, CHUNK), jnp.int32),
            pltpu.VMEM((K * CHUNK, P), jnp.float32),
            pltpu.VMEM((K * CHUNK, P), jnp.float32),
            pltpu.SemaphoreType.DMA,
            pltpu.SemaphoreType.DMA,
            pltpu.SemaphoreType.DMA,
            pltpu.SemaphoreType.DMA,
            pltpu.SemaphoreType.DMA,
            pltpu.SemaphoreType.DMA,
        ],
    )
    def _gather(idx_hbm, table_hbm, out_hbm, idx0, idx1, rows0, rows1,
                semi0, semi1, semg0, semg1, semo0, semo1):
        wid = lax.axis_index("s") * NC + lax.axis_index("c")
        ibase = tok0 * (ROWS_PER_S // CHUNK) + wid * (per_w // CHUNK)
        obase = out_shift + wid * per_w

        def idx_in(g, idx_v, sem):
            g = jnp.minimum(g, groups - 1)
            return pltpu.async_copy(
                idx_hbm.at[pl.ds(ibase + g * K, K)], idx_v, sem
            )

        def idx_wait(idx_v, sem):
            pltpu.make_async_copy(
                idx_hbm.at[pl.ds(ibase, K)], idx_v, sem
            ).wait()

        def fire(idx_v, rows_v, sem):
            for j in range(K):
                pltpu.async_copy(
                    table_hbm.at[idx_v.at[j]],
                    rows_v.at[pl.ds(j * CHUNK, CHUNK)],
                    sem,
                )

        def drain(idx_v, rows_v, sem):
            for j in range(K):
                pltpu.make_async_copy(
                    table_hbm.at[idx_v.at[j]],
                    rows_v.at[pl.ds(j * CHUNK, CHUNK)],
                    sem,
                ).wait()

        def out_start(g, rows_v, sem):
            pltpu.async_copy(
                rows_v,
                out_hbm.at[pl.ds(obase + g * K * CHUNK, K * CHUNK)],
                sem,
            )

        def out_wait(rows_v, sem):
            pltpu.make_async_copy(
                rows_v, out_hbm.at[pl.ds(obase, K * CHUNK)], sem
            ).wait()

        pltpu.sync_copy(idx_hbm.at[pl.ds(ibase, K)], idx0)
        fire(idx0, rows0, semg0)
        idx_in(1, idx1, semi1)
        idx_wait(idx1, semi1)
        fire(idx1, rows1, semg1)
        drain(idx0, rows0, semg0)
        out_start(0, rows0, semo0)
        idx_in(2, idx0, semi0)

        def body(i, carry):
            idx_wait(idx0, semi0)
            out_wait(rows0, semo0)
            fire(idx0, rows0, semg0)
            drain(idx1, rows1, semg1)
            out_start(2 * i - 1, rows1, semo1)
            idx_in(2 * i + 1, idx1, semi1)
            idx_wait(idx1, semi1)
            out_wait(rows1, semo1)
            fire(idx1, rows1, semg1)
            drain(idx0, rows0, semg0)
            out_start(2 * i, rows0, semo0)
            idx_in(2 * i + 2, idx0, semi0)
            return carry

        lax.fori_loop(1, (groups - 1) // 2, body, 0)
        idx_wait(idx0, semi0)
        out_wait(rows0, semo0)
        fire(idx0, rows0, semg0)
        drain(idx1, rows1, semg1)
        out_start(groups - 2, rows1, semo1)
        drain(idx0, rows0, semg0)
        out_start(groups - 1, rows0, semo0)
        out_wait(rows1, semo1)
        out_wait(rows0, semo0)

    return _gather


_sc_gather_a = _make_gather(0, 103, ROWS_PER_S)
_sc_gather_b = _make_gather(103, 97, 0)


def _make_tc_body(with_cls):
    def _tc_body(inp_ref, feat_ref, nw_ref, wnr_ref, wc_ref, pos_ref,
                 cls_ref, gam_ref, bet_ref, out_ref):
        m = lax.dot_general(
            nw_ref[...], wnr_ref[...],
            dimension_numbers=(((1,), (1,)), ((0,), (0,))),
            preferred_element_type=jnp.float32,
        )
        emb = lax.dot_general(
            feat_ref[...], wc_ref[...],
            dimension_numbers=(((2,), (0,)), ((), ())),
            preferred_element_type=jnp.float32,
        )
        emb = emb + lax.dot_general(
            inp_ref[...], m,
            dimension_numbers=(((1,), (0,)), ((), ())),
            preferred_element_type=jnp.float32,
        )
        hid = emb + pos_ref[0][:, None, :]
        mean = jnp.mean(hid, axis=-1, keepdims=True)
        var = jnp.mean((hid - mean) ** 2, axis=-1, keepdims=True)
        body = (hid - mean) * lax.rsqrt(var + 1e-5) * gam_ref[...] \
            + bet_ref[...]
        out_ref[...] = body

        if with_cls:
            @pl.when(pl.program_id(0) == 0)
            def _cls_row():
                row0 = cls_ref[...] + pos_ref[0, 0:1, :]
                m0 = jnp.mean(row0, axis=-1, keepdims=True)
                v0 = jnp.mean((row0 - m0) ** 2, axis=-1, keepdims=True)
                r0 = (row0 - m0) * lax.rsqrt(v0 + 1e-5) * gam_ref[...] \
                    + bet_ref[...]
                out_ref[0:1, :, :] = jnp.broadcast_to(r0[:, None, :],
                                                      (1, B, H))
    return _tc_body


def _dense_a(inp, feat, num_w, wn_r, wc, pos, cls2d, gam, bet):
    return pl.pallas_call(
        _make_tc_body(True),
        grid=(NBLK_A,),
        in_specs=[
            pl.BlockSpec((SS, NUM_COLS, B), lambda i: (i, 0, 0)),
            pl.BlockSpec((SS, B, CAT_COLS * P), lambda i: (i, 0, 0)),
            pl.BlockSpec((NUM_COLS, P), lambda i: (0, 0)),
            pl.BlockSpec((NUM_COLS, P, H), lambda i: (0, 0, 0)),
            pl.BlockSpec((CAT_COLS * P, H), lambda i: (0, 0)),
            pl.BlockSpec((1, SS, H), lambda i: (i, 0, 0)),
            pl.BlockSpec((1, H), lambda i: (0, 0)),
            pl.BlockSpec((1, H), lambda i: (0, 0)),
            pl.BlockSpec((1, H), lambda i: (0, 0)),
        ],
        out_specs=pl.BlockSpec((SS, B, H), lambda i: (i, 0, 0)),
        out_shape=jax.ShapeDtypeStruct((S + 1, B, H), jnp.float32),
    )(inp, feat, num_w, wn_r, wc, pos, cls2d, gam, bet)


def _dense_b(prev, inp, feat, num_w, wn_r, wc, pos, cls2d, gam, bet):
    body = _make_tc_body(False)

    def wrapped(prev_ref, *refs):
        del prev_ref
        body(*refs)

    return pl.pallas_call(
        wrapped,
        grid=(NBLK_A,),
        in_specs=[
            pl.BlockSpec(memory_space=pl.ANY),
            pl.BlockSpec((SS, NUM_COLS, B), lambda i: (i + NBLK_A, 0, 0)),
            pl.BlockSpec((SS, B, CAT_COLS * P), lambda i: (i, 0, 0)),
            pl.BlockSpec((NUM_COLS, P), lambda i: (0, 0)),
            pl.BlockSpec((NUM_COLS, P, H), lambda i: (0, 0, 0)),
            pl.BlockSpec((CAT_COLS * P, H), lambda i: (0, 0)),
            pl.BlockSpec((1, SS, H), lambda i: (i + NBLK_A, 0, 0)),
            pl.BlockSpec((1, H), lambda i: (0, 0)),
            pl.BlockSpec((1, H), lambda i: (0, 0)),
            pl.BlockSpec((1, H), lambda i: (0, 0)),
        ],
        out_specs=pl.BlockSpec((SS, B, H), lambda i: (i + NBLK_A, 0, 0)),
        out_shape=jax.ShapeDtypeStruct((S + 1, B, H), jnp.float32),
        input_output_aliases={0: 0},
    )(prev, inp, feat, num_w, wn_r, wc, pos, cls2d, gam, bet)


def kernel(input, cls_token, pos_table, cat_tables, num_weights, proj_w,
           gamma, beta):
    cat_idx = input[:, :, NUM_COLS:].astype(jnp.int32)
    offs = jnp.arange(CAT_COLS, dtype=jnp.int32) * DICT
    idx = (cat_idx + offs).transpose(1, 0, 2).reshape(N_IDX // CHUNK, CHUNK)
    tbl_phys = cat_tables.reshape(CAT_COLS, VTILES, 128, P)
    tbl_phys = tbl_phys.transpose(0, 1, 3, 2).reshape(TILES_TOTAL * P, 128)
    table_fmt = _sc_tablefmt(tbl_phys)
    table = table_fmt.reshape(CAT_COLS * DICT, P)
    feat_a = _sc_gather_a(idx, table).reshape(SEG_POS, B, CAT_COLS * P)
    feat_b = _sc_gather_b(idx, table).reshape(SEG_POS, B, CAT_COLS * P)
    inp_t = input[:, :, :NUM_COLS].transpose(1, 2, 0)
    inp_pad = jnp.pad(inp_t, ((1, SPAD - S - 1), (0, 0), (0, 0)))
    pos_pad = jnp.pad(pos_table, ((0, SPAD - S - 1), (0, 0)))
    pos_pad = pos_pad.reshape(SPAD // SS, SS, H)
    wn_r = proj_w[:, : NUM_COLS * P].reshape(H, NUM_COLS, P).transpose(1, 2, 0)
    wc = proj_w[:, NUM_COLS * P:].T
    cls2d = cls_token.reshape(1, H)
    gam2d = gamma.reshape(1, H)
    bet2d = beta.reshape(1, H)
    out1 = _dense_a(inp_pad, feat_a, num_weights, wn_r, wc, pos_pad,
                    cls2d, gam2d, bet2d)
    out_sm = _dense_b(out1, inp_pad, feat_b, num_weights, wn_r, wc, pos_pad,
                      cls2d, gam2d, bet2d)
    return out_sm.transpose(1, 0, 2)
